# baseline (device time: 25179 ns/iter reference)
import jax
import jax.numpy as jnp
from jax import lax
from jax.experimental import pallas as pl
from jax.experimental.pallas import tpu as pltpu

N_DEV = 16
BLK = 128
GROUPS = 4
GK = (N_DEV // GROUPS) * BLK


def kernel(x, w_mat):
    k, m_per = x.shape
    k_w, n = w_mat.shape

    def body(x_ref, w_ref, out_ref, gather_ref, send_sems, recv_sems):
        my = lax.axis_index("i")

        barrier_sem = pltpu.get_barrier_semaphore()
        for r in range(1, N_DEV):
            peer = lax.rem(my + r, N_DEV)
            pl.semaphore_signal(
                barrier_sem, inc=1,
                device_id=(peer,), device_id_type=pl.DeviceIdType.MESH,
            )
        pl.semaphore_wait(barrier_sem, N_DEV - 1)

        gather_ref[:, pl.ds(my * BLK, BLK)] = x_ref[pl.ds(my * BLK, BLK), :]

        rdmas = []
        for r in range(1, N_DEV):
            dst = lax.rem(my + r, N_DEV)
            rdma = pltpu.make_async_remote_copy(
                src_ref=x_ref.at[pl.ds(dst * BLK, BLK), :],
                dst_ref=gather_ref.at[:, pl.ds(my * BLK, BLK)],
                send_sem=send_sems.at[r],
                recv_sem=recv_sems.at[my],
                device_id=(dst,),
                device_id_type=pl.DeviceIdType.MESH,
            )
            rdma.start()
            rdmas.append(rdma)

        for g in range(GROUPS):
            for j in range(g * (N_DEV // GROUPS), (g + 1) * (N_DEV // GROUPS)):
                @pl.when(j != my)
                def _():
                    desc = pltpu.make_async_remote_copy(
                        src_ref=x_ref.at[pl.ds(0, BLK), :],
                        dst_ref=gather_ref.at[:, pl.ds(j * BLK, BLK)],
                        send_sem=send_sems.at[0],
                        recv_sem=recv_sems.at[j],
                        device_id=(my,),
                        device_id_type=pl.DeviceIdType.MESH,
                    )
                    desc.wait_recv()
            part = jnp.dot(
                gather_ref[:, pl.ds(g * GK, GK)],
                w_ref[pl.ds(g * GK, GK), :],
                preferred_element_type=jnp.float32,
            )
            if g == 0:
                out_ref[:, :] = part
            else:
                out_ref[:, :] += part

        for rdma in rdmas:
            rdma.wait_send()

        y = out_ref[:, :]
        out_ref[:, :] = y * lax.logistic(y)

    return pl.pallas_call(
        body,
        out_shape=jax.ShapeDtypeStruct((BLK, n), jnp.float32),
        in_specs=[
            pl.BlockSpec(memory_space=pltpu.VMEM),
            pl.BlockSpec(memory_space=pltpu.VMEM),
        ],
        out_specs=pl.BlockSpec(memory_space=pltpu.VMEM),
        scratch_shapes=[
            pltpu.VMEM((BLK, k), jnp.float32),
            pltpu.SemaphoreType.DMA((N_DEV,)),
            pltpu.SemaphoreType.DMA((N_DEV,)),
        ],
        compiler_params=pltpu.CompilerParams(collective_id=0),
    )(x, w_mat)


# device time: 22015 ns/iter; 1.1437x vs baseline; 1.1437x over previous
import jax
import jax.numpy as jnp
from jax import lax
from jax.experimental import pallas as pl
from jax.experimental.pallas import tpu as pltpu

N_DEV = 16
BLK = 128


def kernel(x, w_mat):
    k, m_per = x.shape
    k_w, n = w_mat.shape

    def body(x_ref, w_ref, out_ref, xb_ref, gather_ref, acc_ref,
             send_sems, recv_sems):
        my = lax.axis_index("i")

        xb_ref[:, :] = x_ref[:, :].astype(jnp.bfloat16)

        barrier_sem = pltpu.get_barrier_semaphore()
        for r in range(1, N_DEV):
            peer = lax.rem(my + r, N_DEV)
            pl.semaphore_signal(
                barrier_sem, inc=1,
                device_id=(peer,), device_id_type=pl.DeviceIdType.MESH,
            )
        pl.semaphore_wait(barrier_sem, N_DEV - 1)

        gather_ref[0, :, :] = xb_ref[pl.ds(my * BLK, BLK), :]

        rdmas = []
        for r in range(1, N_DEV):
            dst = lax.rem(my + r, N_DEV)
            rdma = pltpu.make_async_remote_copy(
                src_ref=xb_ref.at[pl.ds(dst * BLK, BLK), :],
                dst_ref=gather_ref.at[r],
                send_sem=send_sems.at[r],
                recv_sem=recv_sems.at[r],
                device_id=(dst,),
                device_id_type=pl.DeviceIdType.MESH,
            )
            rdma.start()
            rdmas.append(rdma)

        out_ref[:, :] = jnp.dot(
            gather_ref[0].astype(jnp.float32),
            w_ref[pl.ds(my * BLK, BLK), :],
            preferred_element_type=jnp.float32,
        )
        rdmas[0].wait_recv()
        j1 = lax.rem(my - 1 + N_DEV, N_DEV)
        acc_ref[:, :] = jnp.dot(
            gather_ref[1].astype(jnp.float32),
            w_ref[pl.ds(j1 * BLK, BLK), :],
            preferred_element_type=jnp.float32,
        )
        for r in range(2, N_DEV):
            rdmas[r - 1].wait_recv()
            j = lax.rem(my - r + N_DEV, N_DEV)
            part = jnp.dot(
                gather_ref[r].astype(jnp.float32),
                w_ref[pl.ds(j * BLK, BLK), :],
                preferred_element_type=jnp.float32,
            )
            if r % 2 == 0:
                out_ref[:, :] += part
            else:
                acc_ref[:, :] += part

        for rdma in rdmas:
            rdma.wait_send()

        y = out_ref[:, :] + acc_ref[:, :]
        out_ref[:, :] = y * lax.logistic(y)

    return pl.pallas_call(
        body,
        out_shape=jax.ShapeDtypeStruct((BLK, n), jnp.float32),
        in_specs=[
            pl.BlockSpec(memory_space=pltpu.VMEM),
            pl.BlockSpec(memory_space=pltpu.VMEM),
        ],
        out_specs=pl.BlockSpec(memory_space=pltpu.VMEM),
        scratch_shapes=[
            pltpu.VMEM((k, m_per), jnp.bfloat16),
            pltpu.VMEM((N_DEV, BLK, BLK), jnp.bfloat16),
            pltpu.VMEM((BLK, n), jnp.float32),
            pltpu.SemaphoreType.DMA((N_DEV,)),
            pltpu.SemaphoreType.DMA((N_DEV,)),
        ],
        compiler_params=pltpu.CompilerParams(collective_id=0),
    )(x, w_mat)
